# baseline (speedup 1.0000x reference)
import jax
import jax.numpy as jnp
from jax.experimental import pallas as pl
from jax.experimental.pallas import tpu as pltpu

N = 131072
C = 256
NUM_CLASSES = 20
BLK = 8192
NBLK = N // BLK


def _mm(a, b):
    return jax.lax.dot_general(
        a, b, (((a.ndim - 1,), (0,)), ((), ())),
        preferred_element_type=jnp.float32)


def _mm_nt(a, b):
    return jax.lax.dot_general(
        a, b, (((1,), (1,)), ((), ())),
        preferred_element_type=jnp.float32)


def _fold(sum_h, g_mat, w, b, gam, bet, n):
    sw = _mm(sum_h, w)
    mu = sw / n + b
    t = _mm(g_mat, w)
    ssq = jnp.sum(w * t, axis=0, keepdims=True)
    ey2 = (ssq + 2.0 * b * sw) / n + b * b
    var = ey2 - mu * mu
    scale = gam * jax.lax.rsqrt(var + 1e-3)
    return w * scale, (b - mu) * scale + bet


def _fused_kernel(feat_ref, seg_ref, gt_ref, inst_ref,
                  wb1_ref, bb1_ref, wb2_ref, bb2_ref, wsegt_ref, bsegt_ref,
                  wxy1_ref, bxy1_ref, gxy_ref, bexy_ref,
                  wz1_ref, bz1_ref, gz_ref, bez_ref,
                  w23t_ref, b23t_ref,
                  res_ref,
                  sumh_ref, g_ref, segnll_ref, vcnt_ref,
                  l1xy_ref, cosxy_ref, l1z_ref, cosz_ref, cnt_ref,
                  hbig_ref, wf_ref, cc_ref):
    i = pl.program_id(0)
    n = jnp.float32(N)

    @pl.when(i == 0)
    def _init():
        sumh_ref[...] = jnp.zeros_like(sumh_ref)
        g_ref[...] = jnp.zeros_like(g_ref)
        segnll_ref[...] = jnp.zeros_like(segnll_ref)
        vcnt_ref[...] = jnp.zeros_like(vcnt_ref)
        l1xy_ref[...] = jnp.zeros_like(l1xy_ref)
        cosxy_ref[...] = jnp.zeros_like(cosxy_ref)
        l1z_ref[...] = jnp.zeros_like(l1z_ref)
        cosz_ref[...] = jnp.zeros_like(cosz_ref)
        cnt_ref[...] = jnp.zeros_like(cnt_ref)

    @pl.when(i < NBLK)
    def _phase1():
        x = feat_ref[...]
        h1 = jnp.maximum(_mm(x, wb1_ref[...]) + bb1_ref[...], 0.0)
        h1 = h1.astype(jnp.bfloat16)
        h = jnp.maximum(_mm(h1, wb2_ref[...]) + bb2_ref[...], 0.0)
        hb = h.astype(jnp.bfloat16)
        hbig_ref[pl.ds(i * BLK, BLK), :] = h.astype(jnp.float8_e4m3fn)

        sumh_ref[...] += jnp.sum(h, axis=0, keepdims=True)
        g_ref[...] += jax.lax.dot_general(
            hb, hb, (((0,), (0,)), ((), ())),
            preferred_element_type=jnp.float32)

        logits = _mm_nt(wsegt_ref[...], hb) + bsegt_ref[...]
        m = jnp.max(logits, axis=0, keepdims=True)
        lse = m + jnp.log(jnp.sum(jnp.exp(logits - m), axis=0,
                                  keepdims=True))
        seg = seg_ref[...]
        seg_c = jnp.clip(seg, 0, NUM_CLASSES - 1)
        rows = jax.lax.broadcasted_iota(jnp.int32, logits.shape, 0)
        picked = jnp.sum(jnp.where(rows == seg_c, logits, 0.0), axis=0,
                         keepdims=True)
        nll = lse - picked
        valid = (seg != -1).astype(jnp.float32)
        segnll_ref[...] += jnp.sum(nll * valid).reshape(1, 1)
        vcnt_ref[...] += jnp.sum(valid).reshape(1, 1)

    @pl.when(i == NBLK)
    def _do_fold():
        wfxy, cxy = _fold(sumh_ref[...], g_ref[...], wxy1_ref[...],
                          bxy1_ref[...], gxy_ref[...], bexy_ref[...], n)
        wfz, cz = _fold(sumh_ref[...], g_ref[...], wz1_ref[...],
                        bz1_ref[...], gz_ref[...], bez_ref[...], n)
        wf_ref[...] = jnp.concatenate([wfxy, wfz],
                                      axis=-1).astype(jnp.bfloat16)
        cc_ref[...] = jnp.concatenate([cxy, cz], axis=-1)

    @pl.when(i >= NBLK)
    def _phase2():
        j = i - NBLK
        hb = hbig_ref[pl.ds(j * BLK, BLK), :].astype(jnp.bfloat16)
        a = jnp.maximum(_mm(hb, wf_ref[...]) + cc_ref[...], 0.0)
        p = _mm_nt(w23t_ref[...], a.astype(jnp.bfloat16)) + b23t_ref[...]

        eye3 = (jax.lax.broadcasted_iota(jnp.int32, (3, 3), 0) ==
                jax.lax.broadcasted_iota(jnp.int32, (3, 3), 1)
                ).astype(jnp.float32)
        gt = _mm_nt(eye3, gt_ref[...])
        mask = (inst_ref[...] != -1).astype(jnp.float32)

        d = jnp.abs(p - gt)
        l1xy = d[0:1, :] + d[1:2, :]
        l1z = d[2:3, :]

        s_pp = p[0:1, :] * p[0:1, :] + p[1:2, :] * p[1:2, :]
        s_gg = gt[0:1, :] * gt[0:1, :] + gt[1:2, :] * gt[1:2, :]
        s_pg = p[0:1, :] * gt[0:1, :] + p[1:2, :] * gt[1:2, :]
        cxy = -s_pg / ((jnp.sqrt(s_pp) + 1e-8) * (jnp.sqrt(s_gg) + 1e-8))

        pz = p[2:3, :]
        gz = gt[2:3, :]
        czv = -(pz * gz) / ((jnp.abs(pz) + 1e-8) * (jnp.abs(gz) + 1e-8))

        l1xy_ref[...] += jnp.sum(l1xy * mask).reshape(1, 1)
        cosxy_ref[...] += jnp.sum(cxy * mask).reshape(1, 1)
        l1z_ref[...] += jnp.sum(l1z * mask).reshape(1, 1)
        cosz_ref[...] += jnp.sum(czv * mask).reshape(1, 1)
        cnt_ref[...] += jnp.sum(mask).reshape(1, 1)

    @pl.when(i == 2 * NBLK - 1)
    def _finalize():
        seg_loss = segnll_ref[0, 0] / jnp.maximum(vcnt_ref[0, 0], 1.0)
        denom = cnt_ref[0, 0] + 1e-8
        bias_xy_l1 = l1xy_ref[0, 0] / denom
        bias_xy_cos = cosxy_ref[0, 0] / denom
        bias_z_l1 = l1z_ref[0, 0] / denom
        bias_z_cos = cosz_ref[0, 0] / denom
        loss = (seg_loss + 2.0 * bias_xy_l1 +
                0.5 * (bias_z_l1 + bias_z_cos))
        res = jnp.stack([loss, seg_loss, bias_xy_l1, bias_xy_cos,
                         bias_z_l1, bias_z_cos])
        res_ref[...] = res.reshape(1, 6)


def _full(shape):
    return pl.BlockSpec(shape, lambda i: tuple(0 for _ in shape))


@jax.jit
def kernel(coord, feat, segment, instance, instance_centroid,
           W_b1, b_b1, W_b2, b_b2,
           W_xy1, b_xy1, g_xy, be_xy, W_xy2, b_xy2,
           W_z1, b_z1, g_z, be_z, W_z2, b_z2,
           W_seg, b_seg):
    f32 = jnp.float32
    bf16 = jnp.bfloat16
    seg2 = segment.astype(jnp.int32).reshape(1, N)
    inst2 = instance.astype(jnp.int32).reshape(1, N)
    row = lambda v: v.reshape(1, -1).astype(f32)
    col = lambda v: v.reshape(-1, 1).astype(f32)

    w23t = jnp.zeros((3, 2 * C), f32)
    w23t = w23t.at[0:2, :C].set(W_xy2.T).at[2:3, C:].set(W_z2.T)
    b23t = jnp.concatenate([b_xy2, b_z2]).reshape(3, 1)

    outs = pl.pallas_call(
        _fused_kernel,
        grid=(2 * NBLK,),
        in_specs=[
            pl.BlockSpec((BLK, 6), lambda i: (i % NBLK, 0)),
            pl.BlockSpec((1, BLK), lambda i: (0, i % NBLK)),
            pl.BlockSpec((BLK, 3), lambda i: (i % NBLK, 0)),
            pl.BlockSpec((1, BLK), lambda i: (0, i % NBLK)),
            _full((6, C)), _full((1, C)), _full((C, C)), _full((1, C)),
            _full((NUM_CLASSES, C)), _full((NUM_CLASSES, 1)),
            _full((C, C)), _full((1, C)), _full((1, C)), _full((1, C)),
            _full((C, C)), _full((1, C)), _full((1, C)), _full((1, C)),
            _full((3, 2 * C)), _full((3, 1)),
        ],
        out_specs=[_full((1, 6))],
        out_shape=[jax.ShapeDtypeStruct((1, 6), f32)],
        scratch_shapes=[
            pltpu.VMEM((1, C), f32), pltpu.VMEM((C, C), f32),
            pltpu.VMEM((1, 1), f32), pltpu.VMEM((1, 1), f32),
            pltpu.VMEM((1, 1), f32), pltpu.VMEM((1, 1), f32),
            pltpu.VMEM((1, 1), f32), pltpu.VMEM((1, 1), f32),
            pltpu.VMEM((1, 1), f32),
            pltpu.VMEM((N, C), jnp.float8_e4m3fn),
            pltpu.VMEM((C, 2 * C), bf16), pltpu.VMEM((1, 2 * C), f32),
        ],
        compiler_params=pltpu.CompilerParams(
            dimension_semantics=("arbitrary",)),
    )(feat, seg2, instance_centroid - coord, inst2,
      W_b1, row(b_b1), W_b2.astype(bf16), row(b_b2),
      W_seg.T.astype(bf16), col(b_seg),
      W_xy1, row(b_xy1), row(g_xy), row(be_xy),
      W_z1, row(b_z1), row(g_z), row(be_z),
      w23t.astype(bf16), b23t)

    return outs[0].reshape(6)


# speedup vs baseline: 1.2286x; 1.2286x over previous
import jax
import jax.numpy as jnp
from jax.experimental import pallas as pl
from jax.experimental.pallas import tpu as pltpu

N = 131072
C = 256
NUM_CLASSES = 20
BLK = 16384


def _mm(a, b):
    return jax.lax.dot_general(
        a, b, (((a.ndim - 1,), (0,)), ((), ())),
        preferred_element_type=jnp.float32)


def _mm_nt(a, b):
    return jax.lax.dot_general(
        a, b, (((1,), (1,)), ((), ())),
        preferred_element_type=jnp.float32)


def _pass1_kernel(feat_ref, seg_ref, wb1_ref, bb1_ref, wb2_ref, bb2_ref,
                  wsegt_ref, bsegt_ref,
                  h_ref, sumh_ref, g_ref, segnll_ref, vcnt_ref):
    i = pl.program_id(0)

    @pl.when(i == 0)
    def _init():
        sumh_ref[...] = jnp.zeros_like(sumh_ref)
        g_ref[...] = jnp.zeros_like(g_ref)
        segnll_ref[...] = jnp.zeros_like(segnll_ref)
        vcnt_ref[...] = jnp.zeros_like(vcnt_ref)

    x = feat_ref[...]
    h1 = jnp.maximum(_mm(x, wb1_ref[...]) + bb1_ref[...], 0.0)
    h1 = h1.astype(jnp.bfloat16)
    h = jnp.maximum(_mm(h1, wb2_ref[...]) + bb2_ref[...], 0.0)
    hb = h.astype(jnp.bfloat16)
    h_ref[...] = h.astype(jnp.float8_e4m3fn)

    sumh_ref[...] += jnp.sum(h, axis=0, keepdims=True)
    g_ref[...] += jax.lax.dot_general(
        hb, hb, (((0,), (0,)), ((), ())), preferred_element_type=jnp.float32)

    logits = _mm_nt(wsegt_ref[...], hb) + bsegt_ref[...]
    m = jnp.max(logits, axis=0, keepdims=True)
    lse = m + jnp.log(jnp.sum(jnp.exp(logits - m), axis=0, keepdims=True))
    seg = seg_ref[...]
    seg_c = jnp.clip(seg, 0, NUM_CLASSES - 1)
    rows = jax.lax.broadcasted_iota(jnp.int32, logits.shape, 0)
    picked = jnp.sum(jnp.where(rows == seg_c, logits, 0.0), axis=0,
                     keepdims=True)
    nll = lse - picked
    valid = (seg != -1).astype(jnp.float32)
    segnll_ref[...] += jnp.sum(nll * valid).reshape(1, 1)
    vcnt_ref[...] += jnp.sum(valid).reshape(1, 1)


def _fold(sum_h, g_mat, w, b, gam, bet, n):
    sw = _mm(sum_h, w)
    mu = sw / n + b
    t = _mm(g_mat, w)
    ssq = jnp.sum(w * t, axis=0, keepdims=True)
    ey2 = (ssq + 2.0 * b * sw) / n + b * b
    var = ey2 - mu * mu
    scale = gam * jax.lax.rsqrt(var + 1e-3)
    return w * scale, (b - mu) * scale + bet


def _pass2_kernel(h_ref, coordt_ref, centt_ref, inst_ref, sumh_ref, g_ref,
                  wxy1_ref, bxy1_ref, gxy_ref, bexy_ref,
                  wz1_ref, bz1_ref, gz_ref, bez_ref,
                  w23t_ref, b23t_ref,
                  l1xy_ref, cosxy_ref, l1z_ref, cosz_ref, cnt_ref,
                  wf_ref, cc_ref):
    i = pl.program_id(0)
    n = jnp.float32(N)

    @pl.when(i == 0)
    def _init():
        wfxy, cxy = _fold(sumh_ref[...], g_ref[...], wxy1_ref[...],
                          bxy1_ref[...], gxy_ref[...], bexy_ref[...], n)
        wfz, cz = _fold(sumh_ref[...], g_ref[...], wz1_ref[...],
                        bz1_ref[...], gz_ref[...], bez_ref[...], n)
        wf_ref[...] = jnp.concatenate([wfxy, wfz], axis=-1).astype(jnp.bfloat16)
        cc_ref[...] = jnp.concatenate([cxy, cz], axis=-1)
        l1xy_ref[...] = jnp.zeros_like(l1xy_ref)
        cosxy_ref[...] = jnp.zeros_like(cosxy_ref)
        l1z_ref[...] = jnp.zeros_like(l1z_ref)
        cosz_ref[...] = jnp.zeros_like(cosz_ref)
        cnt_ref[...] = jnp.zeros_like(cnt_ref)

    hb = h_ref[...].astype(jnp.bfloat16)
    a = jnp.maximum(_mm(hb, wf_ref[...]) + cc_ref[...], 0.0)
    p = _mm_nt(w23t_ref[...], a) + b23t_ref[...]

    gt = centt_ref[...] - coordt_ref[...]
    mask = (inst_ref[...] != -1).astype(jnp.float32)

    d = jnp.abs(p - gt)
    l1xy = d[0:1, :] + d[1:2, :]
    l1z = d[2:3, :]

    s_pp = p[0:1, :] * p[0:1, :] + p[1:2, :] * p[1:2, :]
    s_gg = gt[0:1, :] * gt[0:1, :] + gt[1:2, :] * gt[1:2, :]
    s_pg = p[0:1, :] * gt[0:1, :] + p[1:2, :] * gt[1:2, :]
    cxy = -s_pg / ((jnp.sqrt(s_pp) + 1e-8) * (jnp.sqrt(s_gg) + 1e-8))

    pz = p[2:3, :]
    gz = gt[2:3, :]
    czv = -(pz * gz) / ((jnp.abs(pz) + 1e-8) * (jnp.abs(gz) + 1e-8))

    l1xy_ref[...] += jnp.sum(l1xy * mask).reshape(1, 1)
    cosxy_ref[...] += jnp.sum(cxy * mask).reshape(1, 1)
    l1z_ref[...] += jnp.sum(l1z * mask).reshape(1, 1)
    cosz_ref[...] += jnp.sum(czv * mask).reshape(1, 1)
    cnt_ref[...] += jnp.sum(mask).reshape(1, 1)


def _full(shape):
    return pl.BlockSpec(shape, lambda i: tuple(0 for _ in shape))


@jax.jit
def kernel(coord, feat, segment, instance, instance_centroid,
           W_b1, b_b1, W_b2, b_b2,
           W_xy1, b_xy1, g_xy, be_xy, W_xy2, b_xy2,
           W_z1, b_z1, g_z, be_z, W_z2, b_z2,
           W_seg, b_seg):
    f32 = jnp.float32
    bf16 = jnp.bfloat16
    nblk = N // BLK
    seg2 = segment.astype(jnp.int32).reshape(1, N)
    inst2 = instance.astype(jnp.int32).reshape(1, N)
    coordt = coord.T
    centt = instance_centroid.T
    row = lambda v: v.reshape(1, -1).astype(f32)
    col = lambda v: v.reshape(-1, 1).astype(f32)

    grid = (nblk,)
    params = pltpu.CompilerParams(dimension_semantics=("arbitrary",))

    h, sum_h, g_mat, segnll, vcnt = pl.pallas_call(
        _pass1_kernel,
        grid=grid,
        in_specs=[
            pl.BlockSpec((BLK, 6), lambda i: (i, 0)),
            pl.BlockSpec((1, BLK), lambda i: (0, i)),
            _full((6, C)), _full((1, C)), _full((C, C)), _full((1, C)),
            _full((NUM_CLASSES, C)), _full((NUM_CLASSES, 1)),
        ],
        out_specs=[
            pl.BlockSpec((BLK, C), lambda i: (i, 0)),
            _full((1, C)), _full((C, C)), _full((1, 1)), _full((1, 1)),
        ],
        out_shape=[
            jax.ShapeDtypeStruct((N, C), jnp.float8_e4m3fn),
            jax.ShapeDtypeStruct((1, C), f32),
            jax.ShapeDtypeStruct((C, C), f32),
            jax.ShapeDtypeStruct((1, 1), f32),
            jax.ShapeDtypeStruct((1, 1), f32),
        ],
        compiler_params=params,
    )(feat, seg2, W_b1, row(b_b1), W_b2.astype(bf16), row(b_b2),
      W_seg.T.astype(bf16), col(b_seg))

    w23t = jnp.zeros((3, 2 * C), f32)
    w23t = w23t.at[0:2, :C].set(W_xy2.T).at[2:3, C:].set(W_z2.T)
    b23t = jnp.concatenate([b_xy2, b_z2]).reshape(3, 1)

    l1xy, cosxy, l1z, cosz, cnt = pl.pallas_call(
        _pass2_kernel,
        grid=grid,
        in_specs=[
            pl.BlockSpec((BLK, C), lambda i: (i, 0)),
            pl.BlockSpec((3, BLK), lambda i: (0, i)),
            pl.BlockSpec((3, BLK), lambda i: (0, i)),
            pl.BlockSpec((1, BLK), lambda i: (0, i)),
            _full((1, C)), _full((C, C)),
            _full((C, C)), _full((1, C)), _full((1, C)), _full((1, C)),
            _full((C, C)), _full((1, C)), _full((1, C)), _full((1, C)),
            _full((3, 2 * C)), _full((3, 1)),
        ],
        out_specs=[_full((1, 1))] * 5,
        out_shape=[jax.ShapeDtypeStruct((1, 1), f32)] * 5,
        scratch_shapes=[
            pltpu.VMEM((C, 2 * C), bf16), pltpu.VMEM((1, 2 * C), f32),
        ],
        compiler_params=params,
    )(h, coordt, centt, inst2, sum_h, g_mat,
      W_xy1, row(b_xy1), row(g_xy), row(be_xy),
      W_z1, row(b_z1), row(g_z), row(be_z),
      w23t, b23t)

    seg_loss = segnll[0, 0] / jnp.maximum(vcnt[0, 0], 1.0)
    denom = cnt[0, 0] + 1e-8
    bias_xy_l1 = l1xy[0, 0] / denom
    bias_xy_cos = cosxy[0, 0] / denom
    bias_z_l1 = l1z[0, 0] / denom
    bias_z_cos = cosz[0, 0] / denom
    loss = seg_loss + 2.0 * bias_xy_l1 + 0.5 * (bias_z_l1 + bias_z_cos)
    return jnp.stack([loss, seg_loss, bias_xy_l1, bias_xy_cos,
                      bias_z_l1, bias_z_cos])
